# baseline (speedup 1.0000x reference)
import functools

import jax
import jax.numpy as jnp
from jax.experimental import pallas as pl
from jax.experimental.pallas import tpu as pltpu


def _pick_block(n, target):
    for b in range(min(target, n), 0, -1):
        if n % b == 0 and b % 8 == 0:
            return b
    return n


def _gcn_kernel(x_ref, adj_ref, w1_ref, b1_ref, w2_ref, b2_ref, out_ref,
                s1_ref, s2_ref, *, bm):
    p = pl.program_id(0)
    m = pl.program_id(1)

    @pl.when((p == 0) & (m == 0))
    def _():
        s1_ref[:, :] = jnp.dot(x_ref[:, :], w1_ref[:, :],
                               preferred_element_type=jnp.float32)

    @pl.when(p == 0)
    def _():
        agg = jnp.dot(adj_ref[:, :], s1_ref[:, :],
                      preferred_element_type=jnp.float32)
        h = jnp.maximum(agg + b1_ref[0, :], 0.0)
        s2_ref[pl.ds(m * bm, bm), :] = jnp.dot(
            h, w2_ref[:, :], preferred_element_type=jnp.float32)

    @pl.when(p == 1)
    def _():
        out_ref[:, :] = jnp.dot(adj_ref[:, :], s2_ref[:, :],
                                preferred_element_type=jnp.float32) + b2_ref[0, :]


@jax.jit
def kernel(x, adj, W1, b1, W2, b2):
    n, nfeat = x.shape
    nhid = W1.shape[1]
    nout = W2.shape[1]
    bm = _pick_block(n, 400)
    nm = n // bm

    grid = (2, nm)
    body = functools.partial(_gcn_kernel, bm=bm)
    out = pl.pallas_call(
        body,
        grid=grid,
        in_specs=[
            pl.BlockSpec((n, nfeat), lambda p, m: (0, 0)),
            pl.BlockSpec((bm, n), lambda p, m: (m, 0)),
            pl.BlockSpec((nfeat, nhid), lambda p, m: (0, 0)),
            pl.BlockSpec((1, nhid), lambda p, m: (0, 0)),
            pl.BlockSpec((nhid, nout), lambda p, m: (0, 0)),
            pl.BlockSpec((1, nout), lambda p, m: (0, 0)),
        ],
        out_specs=pl.BlockSpec((bm, nout), lambda p, m: (m, 0)),
        out_shape=jax.ShapeDtypeStruct((n, nout), jnp.float32),
        scratch_shapes=[
            pltpu.VMEM((n, nhid), jnp.float32),
            pltpu.VMEM((n, nout), jnp.float32),
        ],
    )(x, adj, W1, b1.reshape(1, nhid), W2, b2.reshape(1, nout))
    return out
